# baseline (speedup 1.0000x reference)
import numpy as np
import jax
import jax.numpy as jnp
from jax.experimental import pallas as pl
from jax.experimental.pallas import tpu as pltpu

_NUM_EXPERTS = 64
_TOP_K = 8
_R = 8
_H = 512
_W = 512
_CIN = 3
_COUT = 32
_HALO = _R + 2
_K = 2 * (9 * _HALO + 1)


def _convpool_kernel(x_ref, wbig_ref, fcs_ref, fcb_ref, out_ref,
                     xs_ref, acc_ref):
    zrow = jnp.zeros((1, _W), jnp.bfloat16)
    for kw in range(3):
        for i in range(_CIN):
            s = kw * _CIN + i
            plane = x_ref[0, i]
            if kw == 0:
                sh = jnp.concatenate(
                    [jnp.zeros((_H, 1), jnp.float32), plane[:, : _W - 1]], axis=1)
            elif kw == 1:
                sh = plane
            else:
                sh = jnp.concatenate(
                    [plane[:, 1:], jnp.zeros((_H, 1), jnp.float32)], axis=1)
            xs_ref[s, 0:1, :] = zrow
            xs_ref[s, 1:_H + 1, :] = sh.astype(jnp.bfloat16)
            xs_ref[s, _H + 1:_H + 2, :] = zrow

    acc_ref[:] = jnp.zeros((_R * _COUT, _W), jnp.float32)
    ones_row = jnp.ones((1, _W), jnp.bfloat16)

    def body(blk, carry):
        h0 = blk * _R
        slices = [xs_ref[s, pl.ds(h0, _HALO), :] for s in range(9)]
        xcol = jnp.concatenate(
            slices + [ones_row] + slices + [ones_row], axis=0)
        out = jax.lax.dot_general(
            wbig_ref[:], xcol, (((1,), (0,)), ((), ())),
            preferred_element_type=jnp.float32)
        acc_ref[:] += jnp.maximum(out, 0.0)
        return carry

    jax.lax.fori_loop(0, _H // _R, body, 0)

    rs = jax.lax.dot_general(
        jnp.ones((1, _W), jnp.float32), acc_ref[:], (((1,), (1,)), ((), ())),
        preferred_element_type=jnp.float32,
        precision=jax.lax.Precision.HIGHEST)
    logits = jax.lax.dot_general(
        rs, fcs_ref[:], (((1,), (1,)), ((), ())),
        preferred_element_type=jnp.float32,
        precision=jax.lax.Precision.HIGHEST) + fcb_ref[:]
    out_ref[0] = logits


def _topk_kernel(logits_ref, w_ref, i_ref):
    cur = logits_ref[:]
    b = cur.shape[0]
    iota = jax.lax.broadcasted_iota(jnp.int32, (b, _NUM_EXPERTS), 1)
    vals = []
    inds = []
    for _ in range(_TOP_K):
        m = jnp.max(cur, axis=1, keepdims=True)
        idx = jnp.min(jnp.where(cur == m, iota, _NUM_EXPERTS), axis=1,
                      keepdims=True)
        vals.append(m)
        inds.append(idx)
        cur = jnp.where(iota == idx, -jnp.inf, cur)
    v = jnp.concatenate(vals, axis=1)
    e = jnp.exp(v - v[:, 0:1])
    w_ref[:] = e / jnp.sum(e, axis=1, keepdims=True)
    i_ref[:] = jnp.concatenate(inds, axis=1)


def kernel(x, conv_w, conv_b, fc_w, fc_b):
    batch = x.shape[0]
    hw = _H * _W

    D = np.zeros((_R, 3, _HALO), np.float32)
    for r in range(_R):
        for kh in range(3):
            D[r, kh, r + kh] = 1.0
    wbig = jnp.einsum('oihw,rhg->rowig', conv_w, jnp.asarray(D))
    wbig = wbig.reshape(_R * _COUT, 9 * _HALO)
    bias_col = jnp.tile(conv_b, _R).reshape(_R * _COUT, 1)
    wfull = jnp.concatenate([wbig, bias_col], axis=1)
    w_hi = wfull.astype(jnp.bfloat16)
    w_lo = (wfull - w_hi.astype(jnp.float32)).astype(jnp.bfloat16)
    wbig = jnp.concatenate([w_hi, w_lo], axis=1)

    fcs = jnp.tile(fc_w, (1, _R)) * (1.0 / hw)
    fcb_row = fc_b.reshape(1, _NUM_EXPERTS)

    logits = pl.pallas_call(
        _convpool_kernel,
        grid=(batch,),
        in_specs=[
            pl.BlockSpec((1, _CIN, _H, _W), lambda b: (b, 0, 0, 0)),
            pl.BlockSpec((_R * _COUT, _K), lambda b: (0, 0)),
            pl.BlockSpec((_NUM_EXPERTS, _R * _COUT), lambda b: (0, 0)),
            pl.BlockSpec((1, _NUM_EXPERTS), lambda b: (0, 0)),
        ],
        out_specs=pl.BlockSpec((1, 1, _NUM_EXPERTS), lambda b: (b, 0, 0)),
        out_shape=jax.ShapeDtypeStruct((batch, 1, _NUM_EXPERTS), jnp.float32),
        scratch_shapes=[
            pltpu.VMEM((9, _H + 2, _W), jnp.bfloat16),
            pltpu.VMEM((_R * _COUT, _W), jnp.float32),
        ],
    )(x, wbig, fcs, fcb_row)
    logits = logits.reshape(batch, _NUM_EXPERTS)

    weights, indices = pl.pallas_call(
        _topk_kernel,
        out_shape=(
            jax.ShapeDtypeStruct((batch, _TOP_K), jnp.float32),
            jax.ShapeDtypeStruct((batch, _TOP_K), jnp.int32),
        ),
    )(logits)
    return weights, indices


# speedup vs baseline: 1.9131x; 1.9131x over previous
import numpy as np
import jax
import jax.numpy as jnp
from jax.experimental import pallas as pl
from jax.experimental.pallas import tpu as pltpu

_NUM_EXPERTS = 64
_TOP_K = 8
_R = 8
_H = 512
_W = 512
_CIN = 3
_COUT = 32
_HALO = _R + 2
_K = 2 * (9 * _HALO + 1)


def _convpool_kernel(x_ref, wbig_ref, fcs_ref, fcb_ref, out_ref,
                     xs_ref, acc_ref):
    zrow = jnp.zeros((1, _W), jnp.bfloat16)
    for kw in range(3):
        for i in range(_CIN):
            s = kw * _CIN + i
            plane = x_ref[0, i]
            if kw == 0:
                sh = jnp.concatenate(
                    [jnp.zeros((_H, 1), jnp.float32), plane[:, : _W - 1]], axis=1)
            elif kw == 1:
                sh = plane
            else:
                sh = jnp.concatenate(
                    [plane[:, 1:], jnp.zeros((_H, 1), jnp.float32)], axis=1)
            xs_ref[s, 0:1, :] = zrow
            xs_ref[s, 1:_H + 1, :] = sh.astype(jnp.bfloat16)
            xs_ref[s, _H + 1:_H + 2, :] = zrow

    acc_ref[:] = jnp.zeros((_R * _COUT, _W), jnp.float32)
    ones_row = jnp.ones((1, _W), jnp.bfloat16)

    for blk in range(_H // _R):
        h0 = blk * _R
        slices = [xs_ref[s, h0:h0 + _HALO, :] for s in range(9)]
        xcol = jnp.concatenate(
            slices + [ones_row] + slices + [ones_row], axis=0)
        out = jax.lax.dot_general(
            wbig_ref[:], xcol, (((1,), (0,)), ((), ())),
            preferred_element_type=jnp.float32)
        acc_ref[:] += jnp.maximum(out, 0.0)

    rs = jax.lax.dot_general(
        jnp.ones((1, _W), jnp.float32), acc_ref[:], (((1,), (1,)), ((), ())),
        preferred_element_type=jnp.float32,
        precision=jax.lax.Precision.HIGHEST)
    logits = jax.lax.dot_general(
        rs, fcs_ref[:], (((1,), (1,)), ((), ())),
        preferred_element_type=jnp.float32,
        precision=jax.lax.Precision.HIGHEST) + fcb_ref[:]
    out_ref[0] = logits


def _topk_kernel(logits_ref, w_ref, i_ref):
    cur = logits_ref[:]
    b = cur.shape[0]
    iota = jax.lax.broadcasted_iota(jnp.int32, (b, _NUM_EXPERTS), 1)
    vals = []
    inds = []
    for _ in range(_TOP_K):
        m = jnp.max(cur, axis=1, keepdims=True)
        idx = jnp.min(jnp.where(cur == m, iota, _NUM_EXPERTS), axis=1,
                      keepdims=True)
        vals.append(m)
        inds.append(idx)
        cur = jnp.where(iota == idx, -jnp.inf, cur)
    v = jnp.concatenate(vals, axis=1)
    e = jnp.exp(v - v[:, 0:1])
    w_ref[:] = e / jnp.sum(e, axis=1, keepdims=True)
    i_ref[:] = jnp.concatenate(inds, axis=1)


def kernel(x, conv_w, conv_b, fc_w, fc_b):
    batch = x.shape[0]
    hw = _H * _W

    D = np.zeros((_R, 3, _HALO), np.float32)
    for r in range(_R):
        for kh in range(3):
            D[r, kh, r + kh] = 1.0
    wbig = jnp.einsum('oihw,rhg->rowig', conv_w, jnp.asarray(D))
    wbig = wbig.reshape(_R * _COUT, 9 * _HALO)
    bias_col = jnp.tile(conv_b, _R).reshape(_R * _COUT, 1)
    wfull = jnp.concatenate([wbig, bias_col], axis=1)
    w_hi = wfull.astype(jnp.bfloat16)
    w_lo = (wfull - w_hi.astype(jnp.float32)).astype(jnp.bfloat16)
    wbig = jnp.concatenate([w_hi, w_lo], axis=1)

    fcs = jnp.tile(fc_w, (1, _R)) * (1.0 / hw)
    fcb_row = fc_b.reshape(1, _NUM_EXPERTS)

    logits = pl.pallas_call(
        _convpool_kernel,
        grid=(batch,),
        in_specs=[
            pl.BlockSpec((1, _CIN, _H, _W), lambda b: (b, 0, 0, 0)),
            pl.BlockSpec((_R * _COUT, _K), lambda b: (0, 0)),
            pl.BlockSpec((_NUM_EXPERTS, _R * _COUT), lambda b: (0, 0)),
            pl.BlockSpec((1, _NUM_EXPERTS), lambda b: (0, 0)),
        ],
        out_specs=pl.BlockSpec((1, 1, _NUM_EXPERTS), lambda b: (b, 0, 0)),
        out_shape=jax.ShapeDtypeStruct((batch, 1, _NUM_EXPERTS), jnp.float32),
        scratch_shapes=[
            pltpu.VMEM((9, _H + 2, _W), jnp.bfloat16),
            pltpu.VMEM((_R * _COUT, _W), jnp.float32),
        ],
    )(x, wbig, fcs, fcb_row)
    logits = logits.reshape(batch, _NUM_EXPERTS)

    weights, indices = pl.pallas_call(
        _topk_kernel,
        out_shape=(
            jax.ShapeDtypeStruct((batch, _TOP_K), jnp.float32),
            jax.ShapeDtypeStruct((batch, _TOP_K), jnp.int32),
        ),
    )(logits)
    return weights, indices


# speedup vs baseline: 2.8968x; 1.5142x over previous
import numpy as np
import jax
import jax.numpy as jnp
from jax.experimental import pallas as pl
from jax.experimental.pallas import tpu as pltpu

_NUM_EXPERTS = 64
_TOP_K = 8
_R = 8
_H = 512
_W = 512
_CIN = 3
_COUT = 32
_HALO = _R + 2
_K = 9 * _HALO


def _convpool_kernel(x_ref, wbig_ref, negb_ref, cb_ref, fcwt_ref, fcb_ref,
                     out_ref, xs_ref, acc_ref):
    zrow = jnp.zeros((1, _W), jnp.bfloat16)
    for kw in range(3):
        for i in range(_CIN):
            s = kw * _CIN + i
            plane = x_ref[0, i]
            if kw == 0:
                sh = jnp.concatenate(
                    [jnp.zeros((_H, 1), jnp.float32), plane[:, : _W - 1]], axis=1)
            elif kw == 1:
                sh = plane
            else:
                sh = jnp.concatenate(
                    [plane[:, 1:], jnp.zeros((_H, 1), jnp.float32)], axis=1)
            xs_ref[s, 0:1, :] = zrow
            xs_ref[s, 1:_H + 1, :] = sh.astype(jnp.bfloat16)
            xs_ref[s, _H + 1:_H + 2, :] = zrow

    acc_ref[:] = jnp.zeros((_R * _COUT, _W), jnp.float32)

    for blk in range(_H // _R):
        h0 = blk * _R
        xcol = jnp.concatenate(
            [xs_ref[s, h0:h0 + _HALO, :] for s in range(9)], axis=0)
        out = jax.lax.dot_general(
            wbig_ref[:], xcol, (((1,), (0,)), ((), ())),
            preferred_element_type=jnp.float32)
        acc_ref[:] += jnp.maximum(out, negb_ref[:])

    rs = jnp.sum(acc_ref[:], axis=1, keepdims=True)
    fs = rs[0:_COUT]
    for r in range(1, _R):
        fs = fs + rs[r * _COUT:(r + 1) * _COUT]
    feats = fs * jnp.float32(1.0 / (_H * _W)) + cb_ref[:]
    fb = feats.astype(jnp.bfloat16).astype(jnp.float32)
    t = fcwt_ref[:] * fb
    logits = jnp.sum(t, axis=0, keepdims=True) + fcb_ref[:]
    out_ref[0] = logits


def _topk_kernel(logits_ref, w_ref, i_ref):
    cur = logits_ref[:]
    b = cur.shape[0]
    iota = jax.lax.broadcasted_iota(jnp.int32, (b, _NUM_EXPERTS), 1)
    vals = []
    inds = []
    for _ in range(_TOP_K):
        m = jnp.max(cur, axis=1, keepdims=True)
        idx = jnp.min(jnp.where(cur == m, iota, _NUM_EXPERTS), axis=1,
                      keepdims=True)
        vals.append(m)
        inds.append(idx)
        cur = jnp.where(iota == idx, -jnp.inf, cur)
    v = jnp.concatenate(vals, axis=1)
    e = jnp.exp(v - v[:, 0:1])
    w_ref[:] = e / jnp.sum(e, axis=1, keepdims=True)
    i_ref[:] = jnp.concatenate(inds, axis=1)


def kernel(x, conv_w, conv_b, fc_w, fc_b):
    batch = x.shape[0]

    D = np.zeros((_R, 3, _HALO), np.float32)
    for r in range(_R):
        for kh in range(3):
            D[r, kh, r + kh] = 1.0
    wbig = jnp.einsum('oihw,rhg->rowig', conv_w, jnp.asarray(D))
    wbig = wbig.reshape(_R * _COUT, _K).astype(jnp.bfloat16)

    negb = -jnp.tile(conv_b, _R).reshape(_R * _COUT, 1)
    cb = conv_b.reshape(_COUT, 1)
    fcwt = fc_w.T.astype(jnp.bfloat16).astype(jnp.float32)
    fcb_row = fc_b.reshape(1, _NUM_EXPERTS)

    logits = pl.pallas_call(
        _convpool_kernel,
        grid=(batch,),
        in_specs=[
            pl.BlockSpec((1, _CIN, _H, _W), lambda b: (b, 0, 0, 0)),
            pl.BlockSpec((_R * _COUT, _K), lambda b: (0, 0)),
            pl.BlockSpec((_R * _COUT, 1), lambda b: (0, 0)),
            pl.BlockSpec((_COUT, 1), lambda b: (0, 0)),
            pl.BlockSpec((_COUT, _NUM_EXPERTS), lambda b: (0, 0)),
            pl.BlockSpec((1, _NUM_EXPERTS), lambda b: (0, 0)),
        ],
        out_specs=pl.BlockSpec((1, 1, _NUM_EXPERTS), lambda b: (b, 0, 0)),
        out_shape=jax.ShapeDtypeStruct((batch, 1, _NUM_EXPERTS), jnp.float32),
        scratch_shapes=[
            pltpu.VMEM((9, _H + 2, _W), jnp.bfloat16),
            pltpu.VMEM((_R * _COUT, _W), jnp.float32),
        ],
    )(x, wbig, negb, cb, fcwt, fcb_row)
    logits = logits.reshape(batch, _NUM_EXPERTS)

    weights, indices = pl.pallas_call(
        _topk_kernel,
        out_shape=(
            jax.ShapeDtypeStruct((batch, _TOP_K), jnp.float32),
            jax.ShapeDtypeStruct((batch, _TOP_K), jnp.int32),
        ),
    )(logits)
    return weights, indices
